# Initial kernel scaffold; baseline (speedup 1.0000x reference)
import jax
import jax.numpy as jnp
from jax.experimental import pallas as pl


def kernel(x, edge_index, edge_type, rel_embed, graph_ids, Ws):
    raise NotImplementedError("write your pallas kernel here")


# speedup vs baseline: 2.8395x; 2.8395x over previous
import jax
import jax.numpy as jnp
from jax import lax
from jax.experimental import pallas as pl
from jax.experimental.pallas import tpu as pltpu
from jax.experimental.pallas import tpu_sc as plsc

N = 10000
NN = 10240
E = 320000
D = 128
T = 16
G = 64
S = 4
NC = 2
NS = 16
NW = NC * NS
EP = 327680
EPR = EP // 128
ROWS_PER_W = EPR // NW
CHUNK_ROWS = 8
DEGW = 128
BN = 1024
NB = NN // BN


def _tc_build_table(h, rel):
    S_ = h.shape[0]

    def body(h_ref, rel_ref, o_ref):
        hb = h_ref[0]
        o_ref[...] = (hb[None, :, :] * rel_ref[...][:, None, :])[None]

    return pl.pallas_call(
        body,
        grid=(S_, NB),
        in_specs=[
            pl.BlockSpec((1, BN, D), lambda i, j: (i, j, 0)),
            pl.BlockSpec((T, D), lambda i, j: (0, 0)),
        ],
        out_specs=pl.BlockSpec((1, T, BN, D), lambda i, j: (i, 0, j, 0)),
        out_shape=jax.ShapeDtypeStruct((S_, T, NN, D), jnp.float32),
    )(h, rel)


def _tc_cidx(src3, et3):
    NBJ = 20
    BR = EPR // NBJ

    def body(s_ref, e_ref, o_ref):
        i = pl.program_id(0)
        o_ref[...] = (e_ref[...] * NN + s_ref[...] + i * (T * NN))[None]

    return pl.pallas_call(
        body,
        grid=(S, NBJ),
        in_specs=[
            pl.BlockSpec((BR, 128), lambda i, j: (j, 0)),
            pl.BlockSpec((BR, 128), lambda i, j: (j, 0)),
        ],
        out_specs=pl.BlockSpec((1, BR, 128), lambda i, j: (i, j, 0)),
        out_shape=jax.ShapeDtypeStruct((S, EPR, 128), jnp.int32),
    )(src3, et3)


def _sc_pass(n_sub):
    mesh = plsc.VectorSubcoreMesh(core_axis_name="c", subcore_axis_name="s")
    out_type = jax.ShapeDtypeStruct((n_sub, NC, NN, D), jnp.float32)
    ZR = NN // NS
    SR = 32
    NZ = ZR // SR
    scratch = [
        pltpu.VMEM((CHUNK_ROWS, 128), jnp.int32),
        pltpu.VMEM((CHUNK_ROWS, 128), jnp.int32),
        pltpu.VMEM((128, D), jnp.float32),
        pltpu.VMEM((SR, D), jnp.float32),
        pltpu.VMEM_SHARED((NN, D), jnp.float32),
        pltpu.SemaphoreType.DMA,
    ]

    def body(table, idxs, dst3, zacc, out, idx_v, dst_v, rows_v, stage,
             acc, sem):
        c = lax.axis_index("c")
        s = lax.axis_index("s")
        wid = s * NC + c

        for sub in range(n_sub):
            pltpu.sync_copy(zacc, stage)
            for k in range(NZ):
                pltpu.sync_copy(stage, acc.at[pl.ds(s * ZR + k * SR, SR)])
            plsc.subcore_barrier()

            def chunk(g, _):
                row0 = wid * ROWS_PER_W + g * CHUNK_ROWS
                pltpu.sync_copy(idxs.at[sub, pl.ds(row0, CHUNK_ROWS)], idx_v)
                pltpu.sync_copy(dst3.at[pl.ds(row0, CHUNK_ROWS)], dst_v)
                for j in range(CHUNK_ROWS):
                    pltpu.async_copy(table.at[idx_v.at[j]], rows_v, sem).wait()
                    pltpu.sync_copy(rows_v, acc.at[dst_v.at[j]], add=True)
                return 0

            lax.fori_loop(0, ROWS_PER_W // CHUNK_ROWS, chunk, 0)
            plsc.subcore_barrier()
            for k in range(NZ):
                rk = pl.ds(s * ZR + k * SR, SR)
                pltpu.sync_copy(acc.at[rk], stage)
                pltpu.sync_copy(stage, out.at[sub, c, rk])
            plsc.subcore_barrier()

    return pl.kernel(body, out_type=out_type, mesh=mesh,
                     scratch_types=scratch)


def _sc_deg():
    mesh = plsc.VectorSubcoreMesh(core_axis_name="c", subcore_axis_name="s")
    out_type = jax.ShapeDtypeStruct((NC, NN, DEGW), jnp.float32)
    ZR = NN // NS
    SR = 32
    NZ = ZR // SR
    scratch = [
        pltpu.VMEM((CHUNK_ROWS, 128), jnp.int32),
        pltpu.VMEM((128, DEGW), jnp.float32),
        pltpu.VMEM((SR, DEGW), jnp.float32),
        pltpu.VMEM_SHARED((NN, DEGW), jnp.float32),
    ]

    def body(dst3, zdeg, ones_h, deg_out, dst_v, ones_v, dstage, dacc):
        c = lax.axis_index("c")
        s = lax.axis_index("s")
        wid = s * NC + c

        pltpu.sync_copy(zdeg, dstage)
        for k in range(NZ):
            pltpu.sync_copy(dstage, dacc.at[pl.ds(s * ZR + k * SR, SR)])
        pltpu.sync_copy(ones_h, ones_v)
        plsc.subcore_barrier()

        def chunk(g, _):
            row0 = wid * ROWS_PER_W + g * CHUNK_ROWS
            pltpu.sync_copy(dst3.at[pl.ds(row0, CHUNK_ROWS)], dst_v)
            for j in range(CHUNK_ROWS):
                pltpu.sync_copy(ones_v, dacc.at[dst_v.at[j]], add=True)
            return 0

        lax.fori_loop(0, ROWS_PER_W // CHUNK_ROWS, chunk, 0)
        plsc.subcore_barrier()
        for k in range(NZ):
            rk = pl.ds(s * ZR + k * SR, SR)
            pltpu.sync_copy(dacc.at[rk], dstage)
            pltpu.sync_copy(dstage, deg_out.at[c, rk])

    return pl.kernel(body, out_type=out_type, mesh=mesh,
                     scratch_types=scratch)


def _run_sc_deg(dst3, zdeg, ones_h):
    return _sc_deg()(dst3, zdeg, ones_h)


def _run_sc_pass1(xt_flat, cidx1, dst3, zacc):
    return _sc_pass(1)(xt_flat, cidx1, dst3, zacc)


def _run_sc_pass2(t2_flat, cidx4, dst3, zacc):
    return _sc_pass(S)(t2_flat, cidx4, dst3, zacc)


def _tc_mid(p1, pdeg, Ws):
    def body(p_ref, d_ref, w_ref, o_ref):
        p = p_ref[0] + p_ref[1]
        deg = jnp.maximum(d_ref[0, :, 0:1] + d_ref[1, :, 0:1], 1.0)
        agg = p / deg
        h = jnp.maximum(jnp.dot(agg, w_ref[0, 0],
                                preferred_element_type=jnp.float32), 0.0)
        o_ref[...] = h[None]

    return pl.pallas_call(
        body,
        grid=(S, NB),
        in_specs=[
            pl.BlockSpec((NC, BN, D), lambda i, j: (0, j, 0)),
            pl.BlockSpec((NC, BN, DEGW), lambda i, j: (0, j, 0)),
            pl.BlockSpec((1, 1, D, D), lambda i, j: (i, 0, 0, 0)),
        ],
        out_specs=pl.BlockSpec((1, BN, D), lambda i, j: (i, j, 0)),
        out_shape=jax.ShapeDtypeStruct((S, NN, D), jnp.float32),
    )(p1, pdeg, Ws)


def _tc_final(p2, pdeg, Ws, gid3):
    def body(p_ref, d_ref, w_ref, g_ref, o_ref, gs, cs):
        j = pl.program_id(1)

        @pl.when(j == 0)
        def _():
            gs[...] = jnp.zeros((G, D), jnp.float32)
            cs[...] = jnp.zeros((G, D), jnp.float32)

        deg = jnp.maximum(d_ref[0, :, 0:1] + d_ref[1, :, 0:1], 1.0)
        agg = (p_ref[0, 0] + p_ref[0, 1]) / deg
        h2 = jnp.maximum(jnp.dot(agg, w_ref[0, 0],
                                 preferred_element_type=jnp.float32), 0.0)
        gid = g_ref[0, 0, :]
        mask = (gid[None, :] ==
                lax.broadcasted_iota(jnp.int32, (G, BN), 0)
                ).astype(jnp.float32)
        gs[...] += jnp.dot(mask, h2, preferred_element_type=jnp.float32)
        cs[...] = cs[...] + jnp.sum(mask, axis=1, keepdims=True)

        @pl.when(j == NB - 1)
        def _():
            o_ref[...] = (gs[...] / jnp.maximum(cs[...], 1.0))[None]

    return pl.pallas_call(
        body,
        grid=(S, NB),
        in_specs=[
            pl.BlockSpec((1, NC, BN, D), lambda i, j: (i, 0, j, 0)),
            pl.BlockSpec((NC, BN, DEGW), lambda i, j: (0, j, 0)),
            pl.BlockSpec((1, 1, D, D), lambda i, j: (i, 1, 0, 0)),
            pl.BlockSpec((1, 1, BN), lambda i, j: (j, 0, 0)),
        ],
        out_specs=pl.BlockSpec((1, G, D), lambda i, j: (i, 0, 0)),
        out_shape=jax.ShapeDtypeStruct((S, G, D), jnp.float32),
        scratch_shapes=[pltpu.VMEM((G, D), jnp.float32),
                        pltpu.VMEM((G, D), jnp.float32)],
    )(p2, pdeg, Ws, gid3)


def kernel(x, edge_index, edge_type, rel_embed, graph_ids, Ws):
    epad = EP - E
    src = jnp.concatenate([edge_index[0].astype(jnp.int32),
                           jnp.zeros((epad,), jnp.int32)])
    dst = jnp.concatenate([edge_index[1].astype(jnp.int32),
                           jnp.full((epad,), N, jnp.int32)])
    et = jnp.concatenate([edge_type.astype(jnp.int32),
                          jnp.zeros((epad,), jnp.int32)])
    src3 = src.reshape(EPR, 128)
    dst3 = dst.reshape(EPR, 128)
    et3 = et.reshape(EPR, 128)
    xp = jnp.concatenate([x, jnp.zeros((NN - N, D), jnp.float32)])
    gid3 = jnp.concatenate([graph_ids.astype(jnp.int32),
                            jnp.full((NN - N,), G, jnp.int32)]
                           ).reshape(NB, 1, BN)
    zacc = jnp.zeros((32, D), jnp.float32)
    zdeg = jnp.zeros((32, DEGW), jnp.float32)
    ones_h = jnp.ones((128, DEGW), jnp.float32)

    cidx4 = _tc_cidx(src3, et3)
    xt = _tc_build_table(xp[None], rel_embed)
    xt_flat = xt.reshape(T * NN, D)

    pdeg = _run_sc_deg(dst3, zdeg, ones_h)
    p1 = _run_sc_pass1(xt_flat, cidx4[:1], dst3, zacc)
    h = _tc_mid(p1[0], pdeg, Ws)
    t2 = _tc_build_table(h, rel_embed)
    t2_flat = t2.reshape(S * T * NN, D)

    p2 = _run_sc_pass2(t2_flat, cidx4, dst3, zacc)
    out = _tc_final(p2, pdeg, Ws, gid3)
    return out.transpose(1, 0, 2).reshape(G, S * D)


# speedup vs baseline: 3.0295x; 1.0669x over previous
import jax
import jax.numpy as jnp
from jax import lax
from jax.experimental import pallas as pl
from jax.experimental.pallas import tpu as pltpu
from jax.experimental.pallas import tpu_sc as plsc

N = 10000
NN = 10240
E = 320000
D = 128
T = 16
G = 64
S = 4
NC = 2
NS = 16
NW = NC * NS
EP = 327680
EPR = EP // 128
ROWS_PER_W = EPR // NW
CHUNK_ROWS = 8
DEGW = 128
BN = 1024
NB = NN // BN


def _tc_build_table(h, rel):
    S_ = h.shape[0]

    def body(h_ref, rel_ref, o_ref):
        hb = h_ref[0]
        o_ref[...] = (hb[None, :, :] * rel_ref[...][:, None, :])[None]

    return pl.pallas_call(
        body,
        grid=(S_, NB),
        in_specs=[
            pl.BlockSpec((1, BN, D), lambda i, j: (i, j, 0)),
            pl.BlockSpec((T, D), lambda i, j: (0, 0)),
        ],
        out_specs=pl.BlockSpec((1, T, BN, D), lambda i, j: (i, 0, j, 0)),
        out_shape=jax.ShapeDtypeStruct((S_, T, NN, D), jnp.float32),
    )(h, rel)


def _tc_cidx(src3, et3):
    NBJ = 20
    BR = EPR // NBJ

    def body(s_ref, e_ref, o_ref):
        i = pl.program_id(0)
        o_ref[...] = (e_ref[...] * NN + s_ref[...] + i * (T * NN))[None]

    return pl.pallas_call(
        body,
        grid=(S, NBJ),
        in_specs=[
            pl.BlockSpec((BR, 128), lambda i, j: (j, 0)),
            pl.BlockSpec((BR, 128), lambda i, j: (j, 0)),
        ],
        out_specs=pl.BlockSpec((1, BR, 128), lambda i, j: (i, j, 0)),
        out_shape=jax.ShapeDtypeStruct((S, EPR, 128), jnp.int32),
    )(src3, et3)


def _sc_pass(n_sub):
    mesh = plsc.VectorSubcoreMesh(core_axis_name="c", subcore_axis_name="s")
    out_type = jax.ShapeDtypeStruct((n_sub, NC, NN, D), jnp.float32)
    ZR = NN // NS
    SR = 32
    NZ = ZR // SR
    scratch = [
        pltpu.VMEM((CHUNK_ROWS, 128), jnp.int32),
        pltpu.VMEM((CHUNK_ROWS, 128), jnp.int32),
        pltpu.VMEM((2, 128, D), jnp.float32),
        pltpu.VMEM((SR, D), jnp.float32),
        pltpu.VMEM_SHARED((NN, D), jnp.float32),
        pltpu.SemaphoreType.DMA,
        pltpu.SemaphoreType.DMA,
        pltpu.SemaphoreType.DMA,
    ]

    def body(table, idxs, dst3, zacc, out, idx_v, dst_v, rows_v, stage,
             acc, sem, sem_s0, sem_s1):
        c = lax.axis_index("c")
        s = lax.axis_index("s")
        wid = s * NC + c
        ssems = (sem_s0, sem_s1)

        for sub in range(n_sub):
            pltpu.sync_copy(zacc, stage)
            for k in range(NZ):
                pltpu.sync_copy(stage, acc.at[pl.ds(s * ZR + k * SR, SR)])
            plsc.subcore_barrier()

            def chunk(g, _):
                row0 = wid * ROWS_PER_W + g * CHUNK_ROWS
                pltpu.sync_copy(idxs.at[sub, pl.ds(row0, CHUNK_ROWS)], idx_v)
                pltpu.sync_copy(dst3.at[pl.ds(row0, CHUNK_ROWS)], dst_v)
                descs = []
                for j in range(CHUNK_ROWS):
                    b = j % 2
                    if j >= 2:
                        descs[j - 2].wait()
                    pltpu.async_copy(table.at[idx_v.at[j]], rows_v.at[b],
                                     sem).wait()
                    d = pltpu.make_async_copy(rows_v.at[b],
                                              acc.at[dst_v.at[j]], ssems[b])
                    d.start(add=True)
                    descs.append(d)
                descs[-2].wait()
                descs[-1].wait()
                return 0

            lax.fori_loop(0, ROWS_PER_W // CHUNK_ROWS, chunk, 0)
            plsc.subcore_barrier()
            for k in range(NZ):
                rk = pl.ds(s * ZR + k * SR, SR)
                pltpu.sync_copy(acc.at[rk], stage)
                pltpu.sync_copy(stage, out.at[sub, c, rk])
            plsc.subcore_barrier()

    return pl.kernel(body, out_type=out_type, mesh=mesh,
                     scratch_types=scratch)


def _sc_deg():
    mesh = plsc.VectorSubcoreMesh(core_axis_name="c", subcore_axis_name="s")
    out_type = jax.ShapeDtypeStruct((NC, NN, DEGW), jnp.float32)
    ZR = NN // NS
    SR = 32
    NZ = ZR // SR
    scratch = [
        pltpu.VMEM((CHUNK_ROWS, 128), jnp.int32),
        pltpu.VMEM((128, DEGW), jnp.float32),
        pltpu.VMEM((SR, DEGW), jnp.float32),
        pltpu.VMEM_SHARED((NN, DEGW), jnp.float32),
    ]

    def body(dst3, zdeg, ones_h, deg_out, dst_v, ones_v, dstage, dacc):
        c = lax.axis_index("c")
        s = lax.axis_index("s")
        wid = s * NC + c

        pltpu.sync_copy(zdeg, dstage)
        for k in range(NZ):
            pltpu.sync_copy(dstage, dacc.at[pl.ds(s * ZR + k * SR, SR)])
        pltpu.sync_copy(ones_h, ones_v)
        plsc.subcore_barrier()

        def chunk(g, _):
            row0 = wid * ROWS_PER_W + g * CHUNK_ROWS
            pltpu.sync_copy(dst3.at[pl.ds(row0, CHUNK_ROWS)], dst_v)
            for j in range(CHUNK_ROWS):
                pltpu.sync_copy(ones_v, dacc.at[dst_v.at[j]], add=True)
            return 0

        lax.fori_loop(0, ROWS_PER_W // CHUNK_ROWS, chunk, 0)
        plsc.subcore_barrier()
        for k in range(NZ):
            rk = pl.ds(s * ZR + k * SR, SR)
            pltpu.sync_copy(dacc.at[rk], dstage)
            pltpu.sync_copy(dstage, deg_out.at[c, rk])

    return pl.kernel(body, out_type=out_type, mesh=mesh,
                     scratch_types=scratch)


def _run_sc_deg(dst3, zdeg, ones_h):
    return _sc_deg()(dst3, zdeg, ones_h)


def _run_sc_pass1(xt_flat, cidx1, dst3, zacc):
    return _sc_pass(1)(xt_flat, cidx1, dst3, zacc)


def _run_sc_pass2(t2_flat, cidx4, dst3, zacc):
    return _sc_pass(S)(t2_flat, cidx4, dst3, zacc)


def _tc_mid(p1, pdeg, Ws):
    def body(p_ref, d_ref, w_ref, o_ref):
        p = p_ref[0] + p_ref[1]
        deg = jnp.maximum(d_ref[0, :, 0:1] + d_ref[1, :, 0:1], 1.0)
        agg = p / deg
        h = jnp.maximum(jnp.dot(agg, w_ref[0, 0],
                                preferred_element_type=jnp.float32), 0.0)
        o_ref[...] = h[None]

    return pl.pallas_call(
        body,
        grid=(S, NB),
        in_specs=[
            pl.BlockSpec((NC, BN, D), lambda i, j: (0, j, 0)),
            pl.BlockSpec((NC, BN, DEGW), lambda i, j: (0, j, 0)),
            pl.BlockSpec((1, 1, D, D), lambda i, j: (i, 0, 0, 0)),
        ],
        out_specs=pl.BlockSpec((1, BN, D), lambda i, j: (i, j, 0)),
        out_shape=jax.ShapeDtypeStruct((S, NN, D), jnp.float32),
    )(p1, pdeg, Ws)


def _tc_final(p2, pdeg, Ws, gid3):
    def body(p_ref, d_ref, w_ref, g_ref, o_ref, gs, cs):
        j = pl.program_id(1)

        @pl.when(j == 0)
        def _():
            gs[...] = jnp.zeros((G, D), jnp.float32)
            cs[...] = jnp.zeros((G, D), jnp.float32)

        deg = jnp.maximum(d_ref[0, :, 0:1] + d_ref[1, :, 0:1], 1.0)
        agg = (p_ref[0, 0] + p_ref[0, 1]) / deg
        h2 = jnp.maximum(jnp.dot(agg, w_ref[0, 0],
                                 preferred_element_type=jnp.float32), 0.0)
        gid = g_ref[0, 0, :]
        mask = (gid[None, :] ==
                lax.broadcasted_iota(jnp.int32, (G, BN), 0)
                ).astype(jnp.float32)
        gs[...] += jnp.dot(mask, h2, preferred_element_type=jnp.float32)
        cs[...] = cs[...] + jnp.sum(mask, axis=1, keepdims=True)

        @pl.when(j == NB - 1)
        def _():
            o_ref[...] = (gs[...] / jnp.maximum(cs[...], 1.0))[None]

    return pl.pallas_call(
        body,
        grid=(S, NB),
        in_specs=[
            pl.BlockSpec((1, NC, BN, D), lambda i, j: (i, 0, j, 0)),
            pl.BlockSpec((NC, BN, DEGW), lambda i, j: (0, j, 0)),
            pl.BlockSpec((1, 1, D, D), lambda i, j: (i, 1, 0, 0)),
            pl.BlockSpec((1, 1, BN), lambda i, j: (j, 0, 0)),
        ],
        out_specs=pl.BlockSpec((1, G, D), lambda i, j: (i, 0, 0)),
        out_shape=jax.ShapeDtypeStruct((S, G, D), jnp.float32),
        scratch_shapes=[pltpu.VMEM((G, D), jnp.float32),
                        pltpu.VMEM((G, D), jnp.float32)],
    )(p2, pdeg, Ws, gid3)


def kernel(x, edge_index, edge_type, rel_embed, graph_ids, Ws):
    epad = EP - E
    src = jnp.concatenate([edge_index[0].astype(jnp.int32),
                           jnp.zeros((epad,), jnp.int32)])
    dst = jnp.concatenate([edge_index[1].astype(jnp.int32),
                           jnp.full((epad,), N, jnp.int32)])
    et = jnp.concatenate([edge_type.astype(jnp.int32),
                          jnp.zeros((epad,), jnp.int32)])
    src3 = src.reshape(EPR, 128)
    dst3 = dst.reshape(EPR, 128)
    et3 = et.reshape(EPR, 128)
    xp = jnp.concatenate([x, jnp.zeros((NN - N, D), jnp.float32)])
    gid3 = jnp.concatenate([graph_ids.astype(jnp.int32),
                            jnp.full((NN - N,), G, jnp.int32)]
                           ).reshape(NB, 1, BN)
    zacc = jnp.zeros((32, D), jnp.float32)
    zdeg = jnp.zeros((32, DEGW), jnp.float32)
    ones_h = jnp.ones((128, DEGW), jnp.float32)

    cidx4 = _tc_cidx(src3, et3)
    xt = _tc_build_table(xp[None], rel_embed)
    xt_flat = xt.reshape(T * NN, D)

    pdeg = _run_sc_deg(dst3, zdeg, ones_h)
    p1 = _run_sc_pass1(xt_flat, cidx4[:1], dst3, zacc)
    h = _tc_mid(p1[0], pdeg, Ws)
    t2 = _tc_build_table(h, rel_embed)
    t2_flat = t2.reshape(S * T * NN, D)

    p2 = _run_sc_pass2(t2_flat, cidx4, dst3, zacc)
    out = _tc_final(p2, pdeg, Ws, gid3)
    return out.transpose(1, 0, 2).reshape(G, S * D)


# speedup vs baseline: 3.0701x; 1.0134x over previous
import jax
import jax.numpy as jnp
from jax import lax
from jax.experimental import pallas as pl
from jax.experimental.pallas import tpu as pltpu
from jax.experimental.pallas import tpu_sc as plsc

N = 10000
NN = 10240
E = 320000
D = 128
T = 16
G = 64
S = 4
NC = 2
NS = 16
NW = NC * NS
EP = 327680
EPR = EP // 128
ROWS_PER_W = EPR // NW
CHUNK_ROWS = 16
DEGW = 128
BN = 1024
NB = NN // BN


def _tc_build_table(h, rel):
    S_ = h.shape[0]

    def body(h_ref, rel_ref, o_ref):
        hb = h_ref[0]
        o_ref[...] = (hb[None, :, :] * rel_ref[...][:, None, :])[None]

    return pl.pallas_call(
        body,
        grid=(S_, NB),
        in_specs=[
            pl.BlockSpec((1, BN, D), lambda i, j: (i, j, 0)),
            pl.BlockSpec((T, D), lambda i, j: (0, 0)),
        ],
        out_specs=pl.BlockSpec((1, T, BN, D), lambda i, j: (i, 0, j, 0)),
        out_shape=jax.ShapeDtypeStruct((S_, T, NN, D), jnp.float32),
    )(h, rel)


def _tc_cidx(src3, et3):
    NBJ = 20
    BR = EPR // NBJ

    def body(s_ref, e_ref, o_ref):
        i = pl.program_id(0)
        o_ref[...] = (e_ref[...] * NN + s_ref[...] + i * (T * NN))[None]

    return pl.pallas_call(
        body,
        grid=(S, NBJ),
        in_specs=[
            pl.BlockSpec((BR, 128), lambda i, j: (j, 0)),
            pl.BlockSpec((BR, 128), lambda i, j: (j, 0)),
        ],
        out_specs=pl.BlockSpec((1, BR, 128), lambda i, j: (i, j, 0)),
        out_shape=jax.ShapeDtypeStruct((S, EPR, 128), jnp.int32),
    )(src3, et3)


def _sc_pass(n_sub):
    mesh = plsc.VectorSubcoreMesh(core_axis_name="c", subcore_axis_name="s")
    out_type = jax.ShapeDtypeStruct((n_sub, NC, NN, D), jnp.float32)
    ZR = NN // NS
    SR = 32
    NZ = ZR // SR
    scratch = [
        pltpu.VMEM((CHUNK_ROWS, 128), jnp.int32),
        pltpu.VMEM((CHUNK_ROWS, 128), jnp.int32),
        pltpu.VMEM((2, 128, D), jnp.float32),
        pltpu.VMEM((SR, D), jnp.float32),
        pltpu.VMEM_SHARED((NN, D), jnp.float32),
        pltpu.SemaphoreType.DMA,
        pltpu.SemaphoreType.DMA,
        pltpu.SemaphoreType.DMA,
    ]

    def body(table, idxs, dst3, zacc, out, idx_v, dst_v, rows_v, stage,
             acc, sem, sem_s0, sem_s1):
        c = lax.axis_index("c")
        s = lax.axis_index("s")
        wid = s * NC + c
        ssems = (sem_s0, sem_s1)

        for sub in range(n_sub):
            pltpu.sync_copy(zacc, stage)
            for k in range(NZ):
                pltpu.sync_copy(stage, acc.at[pl.ds(s * ZR + k * SR, SR)])
            plsc.subcore_barrier()

            def chunk(g, _):
                row0 = wid * ROWS_PER_W + g * CHUNK_ROWS
                pltpu.sync_copy(idxs.at[sub, pl.ds(row0, CHUNK_ROWS)], idx_v)
                pltpu.sync_copy(dst3.at[pl.ds(row0, CHUNK_ROWS)], dst_v)
                descs = []
                for j in range(CHUNK_ROWS):
                    b = j % 2
                    if j >= 2:
                        descs[j - 2].wait()
                    pltpu.async_copy(table.at[idx_v.at[j]], rows_v.at[b],
                                     sem).wait()
                    d = pltpu.make_async_copy(rows_v.at[b],
                                              acc.at[dst_v.at[j]], ssems[b])
                    d.start(add=True)
                    descs.append(d)
                descs[-2].wait()
                descs[-1].wait()
                return 0

            lax.fori_loop(0, ROWS_PER_W // CHUNK_ROWS, chunk, 0)
            plsc.subcore_barrier()
            for k in range(NZ):
                rk = pl.ds(s * ZR + k * SR, SR)
                pltpu.sync_copy(acc.at[rk], stage)
                pltpu.sync_copy(stage, out.at[sub, c, rk])
            plsc.subcore_barrier()

    return pl.kernel(body, out_type=out_type, mesh=mesh,
                     scratch_types=scratch)


def _sc_deg():
    mesh = plsc.VectorSubcoreMesh(core_axis_name="c", subcore_axis_name="s")
    out_type = jax.ShapeDtypeStruct((NC, NN, DEGW), jnp.float32)
    ZR = NN // NS
    SR = 32
    NZ = ZR // SR
    scratch = [
        pltpu.VMEM((CHUNK_ROWS, 128), jnp.int32),
        pltpu.VMEM((128, DEGW), jnp.float32),
        pltpu.VMEM((SR, DEGW), jnp.float32),
        pltpu.VMEM_SHARED((NN, DEGW), jnp.float32),
    ]

    def body(dst3, zdeg, ones_h, deg_out, dst_v, ones_v, dstage, dacc):
        c = lax.axis_index("c")
        s = lax.axis_index("s")
        wid = s * NC + c

        pltpu.sync_copy(zdeg, dstage)
        for k in range(NZ):
            pltpu.sync_copy(dstage, dacc.at[pl.ds(s * ZR + k * SR, SR)])
        pltpu.sync_copy(ones_h, ones_v)
        plsc.subcore_barrier()

        def chunk(g, _):
            row0 = wid * ROWS_PER_W + g * CHUNK_ROWS
            pltpu.sync_copy(dst3.at[pl.ds(row0, CHUNK_ROWS)], dst_v)
            for j in range(CHUNK_ROWS):
                pltpu.sync_copy(ones_v, dacc.at[dst_v.at[j]], add=True)
            return 0

        lax.fori_loop(0, ROWS_PER_W // CHUNK_ROWS, chunk, 0)
        plsc.subcore_barrier()
        for k in range(NZ):
            rk = pl.ds(s * ZR + k * SR, SR)
            pltpu.sync_copy(dacc.at[rk], dstage)
            pltpu.sync_copy(dstage, deg_out.at[c, rk])

    return pl.kernel(body, out_type=out_type, mesh=mesh,
                     scratch_types=scratch)


def _run_sc_deg(dst3, zdeg, ones_h):
    return _sc_deg()(dst3, zdeg, ones_h)


def _run_sc_pass1(xt_flat, cidx1, dst3, zacc):
    return _sc_pass(1)(xt_flat, cidx1, dst3, zacc)


def _run_sc_pass2(t2_flat, cidx4, dst3, zacc):
    return _sc_pass(S)(t2_flat, cidx4, dst3, zacc)


def _tc_mid(p1, pdeg, Ws):
    def body(p_ref, d_ref, w_ref, o_ref):
        p = p_ref[0] + p_ref[1]
        deg = jnp.maximum(d_ref[0, :, 0:1] + d_ref[1, :, 0:1], 1.0)
        agg = p / deg
        h = jnp.maximum(jnp.dot(agg, w_ref[0, 0],
                                preferred_element_type=jnp.float32), 0.0)
        o_ref[...] = h[None]

    return pl.pallas_call(
        body,
        grid=(S, NB),
        in_specs=[
            pl.BlockSpec((NC, BN, D), lambda i, j: (0, j, 0)),
            pl.BlockSpec((NC, BN, DEGW), lambda i, j: (0, j, 0)),
            pl.BlockSpec((1, 1, D, D), lambda i, j: (i, 0, 0, 0)),
        ],
        out_specs=pl.BlockSpec((1, BN, D), lambda i, j: (i, j, 0)),
        out_shape=jax.ShapeDtypeStruct((S, NN, D), jnp.float32),
    )(p1, pdeg, Ws)


def _tc_final(p2, pdeg, Ws, gid3):
    def body(p_ref, d_ref, w_ref, g_ref, o_ref, gs, cs):
        j = pl.program_id(1)

        @pl.when(j == 0)
        def _():
            gs[...] = jnp.zeros((G, D), jnp.float32)
            cs[...] = jnp.zeros((G, D), jnp.float32)

        deg = jnp.maximum(d_ref[0, :, 0:1] + d_ref[1, :, 0:1], 1.0)
        agg = (p_ref[0, 0] + p_ref[0, 1]) / deg
        h2 = jnp.maximum(jnp.dot(agg, w_ref[0, 0],
                                 preferred_element_type=jnp.float32), 0.0)
        gid = g_ref[0, 0, :]
        mask = (gid[None, :] ==
                lax.broadcasted_iota(jnp.int32, (G, BN), 0)
                ).astype(jnp.float32)
        gs[...] += jnp.dot(mask, h2, preferred_element_type=jnp.float32)
        cs[...] = cs[...] + jnp.sum(mask, axis=1, keepdims=True)

        @pl.when(j == NB - 1)
        def _():
            o_ref[...] = (gs[...] / jnp.maximum(cs[...], 1.0))[None]

    return pl.pallas_call(
        body,
        grid=(S, NB),
        in_specs=[
            pl.BlockSpec((1, NC, BN, D), lambda i, j: (i, 0, j, 0)),
            pl.BlockSpec((NC, BN, DEGW), lambda i, j: (0, j, 0)),
            pl.BlockSpec((1, 1, D, D), lambda i, j: (i, 1, 0, 0)),
            pl.BlockSpec((1, 1, BN), lambda i, j: (j, 0, 0)),
        ],
        out_specs=pl.BlockSpec((1, G, D), lambda i, j: (i, 0, 0)),
        out_shape=jax.ShapeDtypeStruct((S, G, D), jnp.float32),
        scratch_shapes=[pltpu.VMEM((G, D), jnp.float32),
                        pltpu.VMEM((G, D), jnp.float32)],
    )(p2, pdeg, Ws, gid3)


def kernel(x, edge_index, edge_type, rel_embed, graph_ids, Ws):
    epad = EP - E
    src = jnp.concatenate([edge_index[0].astype(jnp.int32),
                           jnp.zeros((epad,), jnp.int32)])
    dst = jnp.concatenate([edge_index[1].astype(jnp.int32),
                           jnp.full((epad,), N, jnp.int32)])
    et = jnp.concatenate([edge_type.astype(jnp.int32),
                          jnp.zeros((epad,), jnp.int32)])
    src3 = src.reshape(EPR, 128)
    dst3 = dst.reshape(EPR, 128)
    et3 = et.reshape(EPR, 128)
    xp = jnp.concatenate([x, jnp.zeros((NN - N, D), jnp.float32)])
    gid3 = jnp.concatenate([graph_ids.astype(jnp.int32),
                            jnp.full((NN - N,), G, jnp.int32)]
                           ).reshape(NB, 1, BN)
    zacc = jnp.zeros((32, D), jnp.float32)
    zdeg = jnp.zeros((32, DEGW), jnp.float32)
    ones_h = jnp.ones((128, DEGW), jnp.float32)

    cidx4 = _tc_cidx(src3, et3)
    xt = _tc_build_table(xp[None], rel_embed)
    xt_flat = xt.reshape(T * NN, D)

    pdeg = _run_sc_deg(dst3, zdeg, ones_h)
    p1 = _run_sc_pass1(xt_flat, cidx4[:1], dst3, zacc)
    h = _tc_mid(p1[0], pdeg, Ws)
    t2 = _tc_build_table(h, rel_embed)
    t2_flat = t2.reshape(S * T * NN, D)

    p2 = _run_sc_pass2(t2_flat, cidx4, dst3, zacc)
    out = _tc_final(p2, pdeg, Ws, gid3)
    return out.transpose(1, 0, 2).reshape(G, S * D)


# speedup vs baseline: 3.1794x; 1.0356x over previous
import jax
import jax.numpy as jnp
from jax import lax
from jax.experimental import pallas as pl
from jax.experimental.pallas import tpu as pltpu
from jax.experimental.pallas import tpu_sc as plsc

N = 10000
NN = 10240
E = 320000
D = 128
T = 16
G = 64
S = 4
NC = 2
NS = 16
NW = NC * NS
EP = 327680
EPR = EP // 128
ROWS_PER_W = EPR // NW
CHUNK_ROWS = 16
DEGW = 128
BN = 1024
NB = NN // BN


def _tc_build_table(h, rel):
    S_ = h.shape[0]

    def body(h_ref, rel_ref, o_ref):
        hb = h_ref[0]
        o_ref[...] = (hb[None, :, :] * rel_ref[...][:, None, :])[None]

    return pl.pallas_call(
        body,
        grid=(S_, NB),
        in_specs=[
            pl.BlockSpec((1, BN, D), lambda i, j: (i, j, 0)),
            pl.BlockSpec((T, D), lambda i, j: (0, 0)),
        ],
        out_specs=pl.BlockSpec((1, T, BN, D), lambda i, j: (i, 0, j, 0)),
        out_shape=jax.ShapeDtypeStruct((S_, T, NN, D), jnp.float32),
    )(h, rel)


def _tc_cidx(src3, et3):
    NBJ = 20
    BR = EPR // NBJ

    def body(s_ref, e_ref, o_ref):
        i = pl.program_id(0)
        o_ref[...] = (e_ref[...] * NN + s_ref[...] + i * (T * NN))[None]

    return pl.pallas_call(
        body,
        grid=(S, NBJ),
        in_specs=[
            pl.BlockSpec((BR, 128), lambda i, j: (j, 0)),
            pl.BlockSpec((BR, 128), lambda i, j: (j, 0)),
        ],
        out_specs=pl.BlockSpec((1, BR, 128), lambda i, j: (i, j, 0)),
        out_shape=jax.ShapeDtypeStruct((S, EPR, 128), jnp.int32),
    )(src3, et3)


def _sc_pass(n_sub):
    mesh = plsc.VectorSubcoreMesh(core_axis_name="c", subcore_axis_name="s")
    out_type = jax.ShapeDtypeStruct((n_sub, NC, NN, D), jnp.float32)
    ZR = NN // NS
    SR = 32
    NZ = ZR // SR
    scratch = [
        pltpu.VMEM((CHUNK_ROWS, 128), jnp.int32),
        pltpu.VMEM((CHUNK_ROWS, 128), jnp.int32),
        pltpu.VMEM((2, 128, D), jnp.float32),
        pltpu.VMEM((SR, D), jnp.float32),
        pltpu.VMEM_SHARED((NN, D), jnp.float32),
        pltpu.SemaphoreType.DMA,
        pltpu.SemaphoreType.DMA,
        pltpu.SemaphoreType.DMA,
        pltpu.SemaphoreType.DMA,
    ]

    def body(table, idxs, dst3, zacc, out, idx_v, dst_v, rows_v, stage,
             acc, sem, sem_g1, sem_s0, sem_s1):
        c = lax.axis_index("c")
        s = lax.axis_index("s")
        wid = s * NC + c
        ssems = (sem_s0, sem_s1)
        gsems = (sem, sem_g1)

        for sub in range(n_sub):
            pltpu.sync_copy(zacc, stage)
            for k in range(NZ):
                pltpu.sync_copy(stage, acc.at[pl.ds(s * ZR + k * SR, SR)])
            plsc.subcore_barrier()

            def chunk(g, _):
                row0 = wid * ROWS_PER_W + g * CHUNK_ROWS
                pltpu.sync_copy(idxs.at[sub, pl.ds(row0, CHUNK_ROWS)], idx_v)
                pltpu.sync_copy(dst3.at[pl.ds(row0, CHUNK_ROWS)], dst_v)
                sdescs = []
                gdescs = []
                for j in range(CHUNK_ROWS):
                    b = j % 2
                    if j >= 2:
                        sdescs[j - 2].wait()
                    gd = pltpu.make_async_copy(table.at[idx_v.at[j]],
                                               rows_v.at[b], gsems[b])
                    gd.start()
                    gdescs.append(gd)
                    if j >= 1:
                        gdescs[j - 1].wait()
                        sd = pltpu.make_async_copy(rows_v.at[(j - 1) % 2],
                                                   acc.at[dst_v.at[j - 1]],
                                                   ssems[(j - 1) % 2])
                        sd.start(add=True)
                        sdescs.append(sd)
                gdescs[-1].wait()
                sd = pltpu.make_async_copy(
                    rows_v.at[(CHUNK_ROWS - 1) % 2],
                    acc.at[dst_v.at[CHUNK_ROWS - 1]],
                    ssems[(CHUNK_ROWS - 1) % 2])
                sd.start(add=True)
                sdescs.append(sd)
                sdescs[-2].wait()
                sdescs[-1].wait()
                return 0

            lax.fori_loop(0, ROWS_PER_W // CHUNK_ROWS, chunk, 0)
            plsc.subcore_barrier()
            for k in range(NZ):
                rk = pl.ds(s * ZR + k * SR, SR)
                pltpu.sync_copy(acc.at[rk], stage)
                pltpu.sync_copy(stage, out.at[sub, c, rk])
            plsc.subcore_barrier()

    return pl.kernel(body, out_type=out_type, mesh=mesh,
                     scratch_types=scratch)


def _sc_deg():
    mesh = plsc.VectorSubcoreMesh(core_axis_name="c", subcore_axis_name="s")
    out_type = jax.ShapeDtypeStruct((NC, NN, DEGW), jnp.float32)
    ZR = NN // NS
    SR = 32
    NZ = ZR // SR
    scratch = [
        pltpu.VMEM((CHUNK_ROWS, 128), jnp.int32),
        pltpu.VMEM((128, DEGW), jnp.float32),
        pltpu.VMEM((SR, DEGW), jnp.float32),
        pltpu.VMEM_SHARED((NN, DEGW), jnp.float32),
    ]

    def body(dst3, zdeg, ones_h, deg_out, dst_v, ones_v, dstage, dacc):
        c = lax.axis_index("c")
        s = lax.axis_index("s")
        wid = s * NC + c

        pltpu.sync_copy(zdeg, dstage)
        for k in range(NZ):
            pltpu.sync_copy(dstage, dacc.at[pl.ds(s * ZR + k * SR, SR)])
        pltpu.sync_copy(ones_h, ones_v)
        plsc.subcore_barrier()

        def chunk(g, _):
            row0 = wid * ROWS_PER_W + g * CHUNK_ROWS
            pltpu.sync_copy(dst3.at[pl.ds(row0, CHUNK_ROWS)], dst_v)
            for j in range(CHUNK_ROWS):
                pltpu.sync_copy(ones_v, dacc.at[dst_v.at[j]], add=True)
            return 0

        lax.fori_loop(0, ROWS_PER_W // CHUNK_ROWS, chunk, 0)
        plsc.subcore_barrier()
        for k in range(NZ):
            rk = pl.ds(s * ZR + k * SR, SR)
            pltpu.sync_copy(dacc.at[rk], dstage)
            pltpu.sync_copy(dstage, deg_out.at[c, rk])

    return pl.kernel(body, out_type=out_type, mesh=mesh,
                     scratch_types=scratch)


def _run_sc_deg(dst3, zdeg, ones_h):
    return _sc_deg()(dst3, zdeg, ones_h)


def _run_sc_pass1(xt_flat, cidx1, dst3, zacc):
    return _sc_pass(1)(xt_flat, cidx1, dst3, zacc)


def _run_sc_pass2(t2_flat, cidx4, dst3, zacc):
    return _sc_pass(S)(t2_flat, cidx4, dst3, zacc)


def _tc_mid(p1, pdeg, Ws):
    def body(p_ref, d_ref, w_ref, o_ref):
        p = p_ref[0] + p_ref[1]
        deg = jnp.maximum(d_ref[0, :, 0:1] + d_ref[1, :, 0:1], 1.0)
        agg = p / deg
        h = jnp.maximum(jnp.dot(agg, w_ref[0, 0],
                                preferred_element_type=jnp.float32), 0.0)
        o_ref[...] = h[None]

    return pl.pallas_call(
        body,
        grid=(S, NB),
        in_specs=[
            pl.BlockSpec((NC, BN, D), lambda i, j: (0, j, 0)),
            pl.BlockSpec((NC, BN, DEGW), lambda i, j: (0, j, 0)),
            pl.BlockSpec((1, 1, D, D), lambda i, j: (i, 0, 0, 0)),
        ],
        out_specs=pl.BlockSpec((1, BN, D), lambda i, j: (i, j, 0)),
        out_shape=jax.ShapeDtypeStruct((S, NN, D), jnp.float32),
    )(p1, pdeg, Ws)


def _tc_final(p2, pdeg, Ws, gid3):
    def body(p_ref, d_ref, w_ref, g_ref, o_ref, gs, cs):
        j = pl.program_id(1)

        @pl.when(j == 0)
        def _():
            gs[...] = jnp.zeros((G, D), jnp.float32)
            cs[...] = jnp.zeros((G, D), jnp.float32)

        deg = jnp.maximum(d_ref[0, :, 0:1] + d_ref[1, :, 0:1], 1.0)
        agg = (p_ref[0, 0] + p_ref[0, 1]) / deg
        h2 = jnp.maximum(jnp.dot(agg, w_ref[0, 0],
                                 preferred_element_type=jnp.float32), 0.0)
        gid = g_ref[0, 0, :]
        mask = (gid[None, :] ==
                lax.broadcasted_iota(jnp.int32, (G, BN), 0)
                ).astype(jnp.float32)
        gs[...] += jnp.dot(mask, h2, preferred_element_type=jnp.float32)
        cs[...] = cs[...] + jnp.sum(mask, axis=1, keepdims=True)

        @pl.when(j == NB - 1)
        def _():
            o_ref[...] = (gs[...] / jnp.maximum(cs[...], 1.0))[None]

    return pl.pallas_call(
        body,
        grid=(S, NB),
        in_specs=[
            pl.BlockSpec((1, NC, BN, D), lambda i, j: (i, 0, j, 0)),
            pl.BlockSpec((NC, BN, DEGW), lambda i, j: (0, j, 0)),
            pl.BlockSpec((1, 1, D, D), lambda i, j: (i, 1, 0, 0)),
            pl.BlockSpec((1, 1, BN), lambda i, j: (j, 0, 0)),
        ],
        out_specs=pl.BlockSpec((1, G, D), lambda i, j: (i, 0, 0)),
        out_shape=jax.ShapeDtypeStruct((S, G, D), jnp.float32),
        scratch_shapes=[pltpu.VMEM((G, D), jnp.float32),
                        pltpu.VMEM((G, D), jnp.float32)],
    )(p2, pdeg, Ws, gid3)


def kernel(x, edge_index, edge_type, rel_embed, graph_ids, Ws):
    epad = EP - E
    src = jnp.concatenate([edge_index[0].astype(jnp.int32),
                           jnp.zeros((epad,), jnp.int32)])
    dst = jnp.concatenate([edge_index[1].astype(jnp.int32),
                           jnp.full((epad,), N, jnp.int32)])
    et = jnp.concatenate([edge_type.astype(jnp.int32),
                          jnp.zeros((epad,), jnp.int32)])
    src3 = src.reshape(EPR, 128)
    dst3 = dst.reshape(EPR, 128)
    et3 = et.reshape(EPR, 128)
    xp = jnp.concatenate([x, jnp.zeros((NN - N, D), jnp.float32)])
    gid3 = jnp.concatenate([graph_ids.astype(jnp.int32),
                            jnp.full((NN - N,), G, jnp.int32)]
                           ).reshape(NB, 1, BN)
    zacc = jnp.zeros((32, D), jnp.float32)
    zdeg = jnp.zeros((32, DEGW), jnp.float32)
    ones_h = jnp.ones((128, DEGW), jnp.float32)

    cidx4 = _tc_cidx(src3, et3)
    xt = _tc_build_table(xp[None], rel_embed)
    xt_flat = xt.reshape(T * NN, D)

    pdeg = _run_sc_deg(dst3, zdeg, ones_h)
    p1 = _run_sc_pass1(xt_flat, cidx4[:1], dst3, zacc)
    h = _tc_mid(p1[0], pdeg, Ws)
    t2 = _tc_build_table(h, rel_embed)
    t2_flat = t2.reshape(S * T * NN, D)

    p2 = _run_sc_pass2(t2_flat, cidx4, dst3, zacc)
    out = _tc_final(p2, pdeg, Ws, gid3)
    return out.transpose(1, 0, 2).reshape(G, S * D)
